# initial kernel scaffold (unmeasured)
import jax
import jax.numpy as jnp
from jax import lax
from jax.experimental import pallas as pl
from jax.experimental.pallas import tpu as pltpu

N = 32
B = 2
SQ = 128
D = 512
HL = 4
DH = 64
SKV_L = 128
QBLK = 64


def kernel(x, Wq, K_ext, V_ext, Wo):
    def body(x_ref, wq_ref, k_ref, v_ref, wo_ref, out_ref,
             k_all, v_all, ar_src, ar_dst,
             k_recv, v_recv, k_send, v_send, ar_send, ar_recv):
        me = lax.axis_index("i")

        k_self = pltpu.make_async_copy(
            k_ref.at[:, :, pl.ds(me * HL, HL), :], k_all.at[me], k_recv.at[me])
        v_self = pltpu.make_async_copy(
            v_ref.at[:, :, pl.ds(me * HL, HL), :], v_all.at[me], v_recv.at[me])
        k_self.start()
        v_self.start()

        bar = pltpu.get_barrier_semaphore()
        for s in range(1, N):
            pl.semaphore_signal(
                bar, inc=1,
                device_id=((me + s) % N,),
                device_id_type=pl.DeviceIdType.MESH,
            )
        pl.semaphore_wait(bar, N - 1)

        k_rdmas = []
        v_rdmas = []
        for s in range(1, N):
            t = (me + s) % N
            rk = pltpu.make_async_remote_copy(
                src_ref=k_ref.at[:, :, pl.ds(t * HL, HL), :],
                dst_ref=k_all.at[me],
                send_sem=k_send.at[t],
                recv_sem=k_recv.at[me],
                device_id=(t,),
                device_id_type=pl.DeviceIdType.MESH,
            )
            rk.start()
            k_rdmas.append(rk)
            rv = pltpu.make_async_remote_copy(
                src_ref=v_ref.at[:, :, pl.ds(t * HL, HL), :],
                dst_ref=v_all.at[me],
                send_sem=v_send.at[t],
                recv_sem=v_recv.at[me],
                device_id=(t,),
                device_id_type=pl.DeviceIdType.MESH,
            )
            rv.start()
            v_rdmas.append(rv)

        q2 = jnp.dot(x_ref[...].reshape(B * SQ, D), wq_ref[...],
                     preferred_element_type=jnp.float32)
        qs = [[q2[b * SQ:(b + 1) * SQ, h * DH:(h + 1) * DH]
               for h in range(HL)] for b in range(B)]

        qb = lax.broadcasted_iota(jnp.int32, (SQ, SKV_L), 0) // QBLK
        col = lax.broadcasted_iota(jnp.int32, (SQ, SKV_L), 1)

        zero_m = jnp.full((SQ, 1), -1e30, jnp.float32)
        zero_l = jnp.zeros((SQ, 1), jnp.float32)
        zero_a = jnp.zeros((SQ, DH), jnp.float32)
        carry0 = tuple((zero_m, zero_l, zero_a) for _ in range(B * HL))

        def step(s, carry):
            j = (me + N - s) % N
            pltpu.make_async_remote_copy(
                src_ref=k_all.at[j], dst_ref=k_all.at[j],
                send_sem=k_send.at[j], recv_sem=k_recv.at[j],
                device_id=(me,), device_id_type=pl.DeviceIdType.MESH,
            ).wait_recv()
            pltpu.make_async_remote_copy(
                src_ref=v_all.at[j], dst_ref=v_all.at[j],
                send_sem=v_send.at[j], recv_sem=v_recv.at[j],
                device_id=(me,), device_id_type=pl.DeviceIdType.MESH,
            ).wait_recv()

            kb = 2 * j + col // QBLK
            mask = (qb == kb) | (kb == 0) | ((qb + kb) % 3 == 0)

            out = []
            for b in range(B):
                for h in range(HL):
                    m, l, acc = carry[b * HL + h]
                    kbh = k_all[j, b, :, h, :]
                    vbh = v_all[j, b, :, h, :]
                    sc = lax.dot_general(
                        qs[b][h], kbh, (((1,), (1,)), ((), ())),
                        preferred_element_type=jnp.float32) * 0.125
                    sc = jnp.where(mask, sc, -1e9)
                    mn = jnp.maximum(m, jnp.max(sc, axis=1, keepdims=True))
                    alpha = jnp.exp(m - mn)
                    p = jnp.exp(sc - mn)
                    ln = l * alpha + jnp.sum(p, axis=1, keepdims=True)
                    pv = lax.dot_general(
                        p, vbh, (((1,), (0,)), ((), ())),
                        preferred_element_type=jnp.float32)
                    out.append((mn, ln, acc * alpha + pv))
            return tuple(out)

        carry = lax.fori_loop(0, N, step, carry0)

        for r in k_rdmas:
            r.wait_send()
        for r in v_rdmas:
            r.wait_send()

        ctx = jnp.concatenate(
            [jnp.concatenate(
                [carry[b * HL + h][2] / carry[b * HL + h][1]
                 for h in range(HL)], axis=1)
             for b in range(B)], axis=0)
        cur = jnp.dot(ctx, wo_ref[...],
                      preferred_element_type=jnp.float32)

        for st in range(5):
            partner = me ^ (1 << st)
            ar_src[...] = cur
            r = pltpu.make_async_remote_copy(
                src_ref=ar_src,
                dst_ref=ar_dst.at[st],
                send_sem=ar_send.at[st],
                recv_sem=ar_recv.at[st],
                device_id=(partner,),
                device_id_type=pl.DeviceIdType.MESH,
            )
            r.start()
            r.wait()
            cur = cur + ar_dst[st]

        out_ref[...] = cur.reshape(B, SQ, D)

    return pl.pallas_call(
        body,
        out_shape=jax.ShapeDtypeStruct((B, SQ, D), jnp.float32),
        in_specs=[pl.BlockSpec(memory_space=pltpu.VMEM)] * 5,
        out_specs=pl.BlockSpec(memory_space=pltpu.VMEM),
        scratch_shapes=[
            pltpu.VMEM((N, B, SKV_L, HL, DH), jnp.float32),
            pltpu.VMEM((N, B, SKV_L, HL, DH), jnp.float32),
            pltpu.VMEM((B * SQ, D), jnp.float32),
            pltpu.VMEM((5, B * SQ, D), jnp.float32),
            pltpu.SemaphoreType.DMA((N,)),
            pltpu.SemaphoreType.DMA((N,)),
            pltpu.SemaphoreType.DMA((N,)),
            pltpu.SemaphoreType.DMA((N,)),
            pltpu.SemaphoreType.DMA((5,)),
            pltpu.SemaphoreType.DMA((5,)),
        ],
        compiler_params=pltpu.CompilerParams(collective_id=0),
    )(x, Wq, K_ext, V_ext, Wo)


# baseline (device time: 562148 ns/iter reference)
import jax
import jax.numpy as jnp
from jax import lax
from jax.experimental import pallas as pl
from jax.experimental.pallas import tpu as pltpu

N = 32
B = 2
SQ = 128
D = 512
HL = 4
DH = 64
SKV_L = 128
QBLK = 64


def kernel(x, Wq, K_ext, V_ext, Wo):
    def body(x_ref, wq_ref, k_ref, v_ref, wo_ref, out_ref,
             k_all, v_all, ar_src, ar_dst,
             k_recv, v_recv, k_send, v_send, ar_send, ar_recv):
        me = lax.axis_index("i")

        k_self = pltpu.make_async_copy(
            k_ref.at[:, :, pl.ds(me * HL, HL), :], k_all.at[0], k_recv.at[0])
        v_self = pltpu.make_async_copy(
            v_ref.at[:, :, pl.ds(me * HL, HL), :], v_all.at[0], v_recv.at[0])
        k_self.start()
        v_self.start()

        k_rdmas = []
        v_rdmas = []
        for s in range(1, N):
            t = (me + s) % N
            rk = pltpu.make_async_remote_copy(
                src_ref=k_ref.at[:, :, pl.ds(t * HL, HL), :],
                dst_ref=k_all.at[s],
                send_sem=k_send.at[s],
                recv_sem=k_recv.at[s],
                device_id=(t,),
                device_id_type=pl.DeviceIdType.MESH,
            )
            rk.start()
            k_rdmas.append(rk)
            rv = pltpu.make_async_remote_copy(
                src_ref=v_ref.at[:, :, pl.ds(t * HL, HL), :],
                dst_ref=v_all.at[s],
                send_sem=v_send.at[s],
                recv_sem=v_recv.at[s],
                device_id=(t,),
                device_id_type=pl.DeviceIdType.MESH,
            )
            rv.start()
            v_rdmas.append(rv)

        q2 = jnp.dot(x_ref[...].reshape(B * SQ, D), wq_ref[...],
                     preferred_element_type=jnp.float32)
        qs = [[q2[b * SQ:(b + 1) * SQ, h * DH:(h + 1) * DH]
               for h in range(HL)] for b in range(B)]

        qb = lax.broadcasted_iota(jnp.int32, (SQ, SKV_L), 0) // QBLK
        col = lax.broadcasted_iota(jnp.int32, (SQ, SKV_L), 1)

        zero_m = jnp.full((SQ, 1), -1e30, jnp.float32)
        zero_l = jnp.zeros((SQ, 1), jnp.float32)
        zero_a = jnp.zeros((SQ, DH), jnp.float32)
        carry = [(zero_m, zero_l, zero_a) for _ in range(B * HL)]

        def process_chunk(slot, j, carry):
            kb = 2 * j + col // QBLK
            mask = (qb == kb) | (kb == 0) | ((qb + kb) % 3 == 0)
            out = []
            for b in range(B):
                for h in range(HL):
                    m, l, acc = carry[b * HL + h]
                    kbh = k_all[slot, b, :, h, :]
                    vbh = v_all[slot, b, :, h, :]
                    sc = lax.dot_general(
                        qs[b][h], kbh, (((1,), (1,)), ((), ())),
                        preferred_element_type=jnp.float32) * 0.125
                    sc = jnp.where(mask, sc, -1e9)
                    mn = jnp.maximum(m, jnp.max(sc, axis=1, keepdims=True))
                    alpha = jnp.exp(m - mn)
                    p = jnp.exp(sc - mn)
                    ln = l * alpha + jnp.sum(p, axis=1, keepdims=True)
                    pv = lax.dot_general(
                        p, vbh, (((1,), (0,)), ((), ())),
                        preferred_element_type=jnp.float32)
                    out.append((mn, ln, acc * alpha + pv))
            return out

        k_self.wait()
        v_self.wait()
        carry = process_chunk(0, me, carry)
        for s in range(1, N):
            k_rdmas[s - 1].wait_recv()
            v_rdmas[s - 1].wait_recv()
            j = (me + N - s) % N
            carry = process_chunk(s, j, carry)

        for r in k_rdmas:
            r.wait_send()
        for r in v_rdmas:
            r.wait_send()

        ctx = jnp.concatenate(
            [jnp.concatenate(
                [carry[b * HL + h][2] / carry[b * HL + h][1]
                 for h in range(HL)], axis=1)
             for b in range(B)], axis=0)
        cur = jnp.dot(ctx, wo_ref[...],
                      preferred_element_type=jnp.float32)

        for st in range(5):
            partner = me ^ (1 << st)
            ar_src[...] = cur
            r = pltpu.make_async_remote_copy(
                src_ref=ar_src,
                dst_ref=ar_dst.at[st],
                send_sem=ar_send.at[st],
                recv_sem=ar_recv.at[st],
                device_id=(partner,),
                device_id_type=pl.DeviceIdType.MESH,
            )
            r.start()
            r.wait()
            cur = cur + ar_dst[st]

        out_ref[...] = cur.reshape(B, SQ, D)

    return pl.pallas_call(
        body,
        out_shape=jax.ShapeDtypeStruct((B, SQ, D), jnp.float32),
        in_specs=[
            pl.BlockSpec(memory_space=pltpu.VMEM),
            pl.BlockSpec(memory_space=pltpu.VMEM),
            pl.BlockSpec(memory_space=pl.ANY),
            pl.BlockSpec(memory_space=pl.ANY),
            pl.BlockSpec(memory_space=pltpu.VMEM),
        ],
        out_specs=pl.BlockSpec(memory_space=pltpu.VMEM),
        scratch_shapes=[
            pltpu.VMEM((N, B, SKV_L, HL, DH), jnp.float32),
            pltpu.VMEM((N, B, SKV_L, HL, DH), jnp.float32),
            pltpu.VMEM((B * SQ, D), jnp.float32),
            pltpu.VMEM((5, B * SQ, D), jnp.float32),
            pltpu.SemaphoreType.DMA((N,)),
            pltpu.SemaphoreType.DMA((N,)),
            pltpu.SemaphoreType.DMA((N,)),
            pltpu.SemaphoreType.DMA((N,)),
            pltpu.SemaphoreType.DMA((5,)),
            pltpu.SemaphoreType.DMA((5,)),
        ],
        compiler_params=pltpu.CompilerParams(
            vmem_limit_bytes=60 * 1024 * 1024,
        ),
    )(x, Wq, K_ext, V_ext, Wo)
